# Initial kernel scaffold; baseline (speedup 1.0000x reference)
import jax
import jax.numpy as jnp
from jax.experimental import pallas as pl


def kernel(X, pair_indices, d_ij, radial_feature_vector, atomic_charges, W1, b1, W2, b2, W3, b3, L0, L1, L2, L3, L4, L5):
    raise NotImplementedError("write your pallas kernel here")


# speedup vs baseline: 16.1934x; 16.1934x over previous
import functools

import jax
import jax.numpy as jnp
from jax import lax
from jax.experimental import pallas as pl
from jax.experimental.pallas import tpu as pltpu
from jax.experimental.pallas import tpu_sc as plsc

N_ATOMS = 10000
N_FEAT = 128
N_RBF = 16
N_EDGES = 160000
CUTOFF = 5.0

NCORE = 2
NSUB = 16
NCH = 8
CPC = NCH // NCORE
EB = 128
NB = 79
EPT = NB * EB
E_PAD = EPT * NSUB
NPT = N_ATOMS // NSUB

_MLP_BLK = 1024
_NODE_BLK = 400


def _silu(x):
    return x * jax.nn.sigmoid(x)


def _mlp_body(rfv_ref, d_ref, w1_ref, b1_ref, w2_ref, b2_ref, w3_ref, b3_ref,
              out_ref):
    h = _silu(jnp.dot(rfv_ref[...], w1_ref[...],
                      preferred_element_type=jnp.float32) + b1_ref[...])
    h = _silu(jnp.dot(h, w2_ref[...],
                      preferred_element_type=jnp.float32) + b2_ref[...])
    h = _silu(jnp.dot(h, w3_ref[...],
                      preferred_element_type=jnp.float32) + b3_ref[...])
    d = d_ref[...]
    c = jnp.where(d < CUTOFF,
                  0.5 * (jnp.cos((jnp.pi / CUTOFF) * d) + 1.0), 0.0)
    out_ref[...] = h * c


def _mlp_call(rfv, d, w1, b1, w2, b2, w3p, b3p):
    grid = E_PAD // _MLP_BLK
    return pl.pallas_call(
        _mlp_body,
        grid=(grid,),
        in_specs=[
            pl.BlockSpec((_MLP_BLK, N_RBF), lambda i: (i, 0)),
            pl.BlockSpec((_MLP_BLK, 1), lambda i: (i, 0)),
            pl.BlockSpec((N_RBF, N_FEAT), lambda i: (0, 0)),
            pl.BlockSpec((1, N_FEAT), lambda i: (0, 0)),
            pl.BlockSpec((N_FEAT, 2 * N_FEAT), lambda i: (0, 0)),
            pl.BlockSpec((1, 2 * N_FEAT), lambda i: (0, 0)),
            pl.BlockSpec((2 * N_FEAT, 3 * N_FEAT), lambda i: (0, 0)),
            pl.BlockSpec((1, 3 * N_FEAT), lambda i: (0, 0)),
        ],
        out_specs=pl.BlockSpec((_MLP_BLK, 3 * N_FEAT), lambda i: (i, 0)),
        out_shape=jax.ShapeDtypeStruct((E_PAD, 3 * N_FEAT), jnp.float32),
        compiler_params=pltpu.CompilerParams(
            dimension_semantics=("parallel",)),
    )(rfv, d, w1, b1, w2, b2, w3p, b3p)


def _prep_body(x_ref, l0_ref, l1_ref, l2_ref, u_ref, xn_ref):
    xs = [x_ref[k] for k in range(9)]
    nrm = xs[0] * xs[0]
    for k in range(1, 9):
        nrm = nrm + xs[k] * xs[k]
    inv = 1.0 / (nrm + 1.0)
    xn = [v * inv for v in xs]
    for k in range(9):
        xn_ref[k] = xn[k]
    dg = (xn[0] + xn[4] + xn[8]) * (1.0 / 3.0)
    up = [dg,
          0.5 * (xn[1] - xn[3]), 0.5 * (xn[2] - xn[6]), 0.5 * (xn[5] - xn[7]),
          0.5 * (xn[1] + xn[3]), 0.5 * (xn[2] + xn[6]), 0.5 * (xn[5] + xn[7]),
          xn[0] - dg, xn[4] - dg]
    u_ref[0] = jnp.dot(up[0], l0_ref[...], preferred_element_type=jnp.float32)
    for k in range(1, 4):
        u_ref[k] = jnp.dot(up[k], l1_ref[...],
                           preferred_element_type=jnp.float32)
    for k in range(4, 9):
        u_ref[k] = jnp.dot(up[k], l2_ref[...],
                           preferred_element_type=jnp.float32)


def _prep_call(xr, l0, l1, l2):
    grid = N_ATOMS // _NODE_BLK
    lmat = lambda: pl.BlockSpec((N_FEAT, N_FEAT), lambda i: (0, 0))
    tens = lambda: pl.BlockSpec((9, _NODE_BLK, N_FEAT), lambda i: (0, i, 0))
    return pl.pallas_call(
        _prep_body,
        grid=(grid,),
        in_specs=[tens(), lmat(), lmat(), lmat()],
        out_specs=[tens(), tens()],
        out_shape=[jax.ShapeDtypeStruct((9, N_ATOMS, N_FEAT), jnp.float32),
                   jax.ShapeDtypeStruct((9, N_ATOMS, N_FEAT), jnp.float32)],
        compiler_params=pltpu.CompilerParams(
            dimension_semantics=("parallel",)),
    )(xr, l0, l1, l2)


def _sc_body(table_ref, w_ref, src_ref, dst_ref, out_ref,
             srcb, dstb, gbuf, wbuf, zbuf, acc, gsem):
    c = lax.axis_index("c")
    s = lax.axis_index("s")
    e0 = s * EPT
    base = s * NPT

    def zrow(i, carry):
        for k in range(9):
            zbuf[i, k] = jnp.zeros((16,), jnp.float32)
        return carry
    lax.fori_loop(0, 25, zrow, 0)

    for cl in range(CPC):
        chunk = c * CPC + cl
        for z in range(25):
            pltpu.sync_copy(zbuf, acc.at[pl.ds(base + z * 25, 25)])
        plsc.subcore_barrier()

        def ebatch(b, carry):
            pltpu.sync_copy(src_ref.at[chunk, s, b], srcb)
            pltpu.sync_copy(dst_ref.at[s, b], dstb)
            pltpu.sync_copy(w_ref.at[chunk, pl.ds(e0 + b * EB, EB)], wbuf)
            pltpu.async_copy(table_ref.at[srcb.at[0]], gbuf, gsem).wait()

            def edge(e, cc):
                w0 = wbuf[e, pl.ds(0, 16)]
                w1 = wbuf[e, pl.ds(16, 16)]
                w2 = wbuf[e, pl.ds(32, 16)]
                gbuf[e, 0] = gbuf[e, 0] * w0
                for k in (1, 2, 3):
                    gbuf[e, k] = gbuf[e, k] * w1
                for k in (4, 5, 6, 7, 8):
                    gbuf[e, k] = gbuf[e, k] * w2
                return cc
            lax.fori_loop(0, EB, edge, 0)
            pltpu.sync_copy(gbuf, acc.at[dstb.at[0]], add=True)
            return carry
        lax.fori_loop(0, NB, ebatch, 0)
        plsc.subcore_barrier()
        pltpu.sync_copy(acc.at[pl.ds(base, NPT)],
                        out_ref.at[chunk, pl.ds(base, NPT)])
        plsc.subcore_barrier()


def _sc_call(table, w, src_hbm, dst_hbm):
    mesh = plsc.VectorSubcoreMesh(core_axis_name="c", subcore_axis_name="s")
    fn = pl.kernel(
        _sc_body,
        out_type=jax.ShapeDtypeStruct((NCH, N_ATOMS, 9, 16), jnp.float32),
        mesh=mesh,
        scratch_types=[
            pltpu.VMEM((1, EB), jnp.int32),
            pltpu.VMEM((1, EB), jnp.int32),
            pltpu.VMEM((EB, 9, 16), jnp.float32),
            pltpu.VMEM((EB, 48), jnp.float32),
            pltpu.VMEM((25, 9, 16), jnp.float32),
            pltpu.VMEM_SHARED((N_ATOMS, 9, 16), jnp.float32),
            pltpu.SemaphoreType.DMA,
        ],
        compiler_params=pltpu.CompilerParams(use_tc_tiling_on_sc=False),
    )
    return fn(table, w, src_hbm, dst_hbm)


def _decompress(u):
    return [u[0] + u[7], u[1] + u[4], u[2] + u[5],
            u[4] - u[1], u[0] + u[8], u[3] + u[6],
            u[5] - u[2], u[6] - u[3], u[0] - u[7] - u[8]]


def _combine_body(msg_ref, u_ref, xn_ref, q_ref, l3_ref, l4_ref, l5_ref,
                  out_ref):
    m = _decompress([msg_ref[k] for k in range(9)])
    y = _decompress([u_ref[k] for k in range(9)])
    t = [None] * 9
    for i in range(3):
        for k in range(3):
            acc = None
            for j in range(3):
                term = m[3 * i + j] * y[3 * j + k] + y[3 * i + j] * m[3 * j + k]
                acc = term if acc is None else acc + term
            t[3 * i + k] = acc
    nrm = t[0] * t[0]
    for k in range(1, 9):
        nrm = nrm + t[k] * t[k]
    inv = 1.0 / (nrm + 1.0)
    dg = (t[0] + t[4] + t[8]) * (1.0 / 3.0)
    v = [dg,
         0.5 * (t[1] - t[3]), 0.5 * (t[2] - t[6]), 0.5 * (t[5] - t[7]),
         0.5 * (t[1] + t[3]), 0.5 * (t[2] + t[6]), 0.5 * (t[5] + t[7]),
         t[0] - dg, t[4] - dg]
    v = [vi * inv for vi in v]
    w = [jnp.dot(v[0], l3_ref[...], preferred_element_type=jnp.float32)]
    for k in range(1, 4):
        w.append(jnp.dot(v[k], l4_ref[...],
                         preferred_element_type=jnp.float32))
    for k in range(4, 9):
        w.append(jnp.dot(v[k], l5_ref[...],
                         preferred_element_type=jnp.float32))
    dx = _decompress(w)
    f = 1.0 + 0.1 * q_ref[...]
    for i in range(3):
        for k in range(3):
            acc = None
            for j in range(3):
                term = dx[3 * i + j] * dx[3 * j + k]
                acc = term if acc is None else acc + term
            out_ref[3 * i + k] = (xn_ref[3 * i + k] + dx[3 * i + k]
                                  + f * acc)


def _combine_call(msg, u_lin, xn, q, l3, l4, l5):
    grid = N_ATOMS // _NODE_BLK
    lmat = lambda: pl.BlockSpec((N_FEAT, N_FEAT), lambda i: (0, 0))
    tens = lambda: pl.BlockSpec((9, _NODE_BLK, N_FEAT), lambda i: (0, i, 0))
    return pl.pallas_call(
        _combine_body,
        grid=(grid,),
        in_specs=[tens(), tens(), tens(),
                  pl.BlockSpec((_NODE_BLK, 1), lambda i: (i, 0)),
                  lmat(), lmat(), lmat()],
        out_specs=tens(),
        out_shape=jax.ShapeDtypeStruct((9, N_ATOMS, N_FEAT), jnp.float32),
        compiler_params=pltpu.CompilerParams(
            dimension_semantics=("parallel",)),
    )(msg, u_lin, xn, q, l3, l4, l5)


def kernel(X, pair_indices, d_ij, radial_feature_vector, atomic_charges,
           W1, b1, W2, b2, W3, b3, L0, L1, L2, L3, L4, L5):
    cols = jnp.arange(3 * N_FEAT)
    cchunk = cols // 48
    rem = cols % 48
    chan = rem // 16
    feat = rem % 16
    src_col = (cchunk * 16 + feat) * 3 + chan
    w3p = W3[:, src_col]
    b3p = b3[src_col]

    pad = E_PAD - N_EDGES
    rfv_p = jnp.pad(radial_feature_vector, ((0, pad), (0, 0)))
    d_p = jnp.pad(d_ij, ((0, pad), (0, 0)), constant_values=CUTOFF)
    pi = pair_indices.astype(jnp.int32)
    dst = jnp.pad(pi[0], (0, pad))
    src = jnp.pad(pi[1], (0, pad))
    src_shift = src[None, :] + (jnp.arange(NCH, dtype=jnp.int32)
                                * N_ATOMS)[:, None]
    src_hbm = src_shift.reshape(NCH, NSUB, NB, 1, EB)
    dst_hbm = dst.reshape(NSUB, NB, 1, EB)

    w = _mlp_call(rfv_p, d_p, W1, b1.reshape(1, -1), W2, b2.reshape(1, -1),
                  w3p, b3p.reshape(1, -1))
    w = w.reshape(E_PAD, NCH, 48).transpose(1, 0, 2)

    xr = X.transpose(2, 3, 0, 1).reshape(9, N_ATOMS, N_FEAT)
    u_lin, xn = _prep_call(xr, L0, L1, L2)

    table = (u_lin.reshape(9, N_ATOMS, NCH, 16)
             .transpose(2, 1, 0, 3)
             .reshape(NCH * N_ATOMS, 9, 16))
    msg_chunks = _sc_call(table, w, src_hbm, dst_hbm)
    msg = (msg_chunks.transpose(2, 1, 0, 3)
           .reshape(9, N_ATOMS, N_FEAT))

    xout = _combine_call(msg, u_lin, xn, atomic_charges.reshape(N_ATOMS, 1),
                         L3, L4, L5)
    return xout.reshape(3, 3, N_ATOMS, N_FEAT).transpose(2, 3, 0, 1)


# speedup vs baseline: 19.3048x; 1.1921x over previous
import functools

import jax
import jax.numpy as jnp
from jax import lax
from jax.experimental import pallas as pl
from jax.experimental.pallas import tpu as pltpu
from jax.experimental.pallas import tpu_sc as plsc

N_ATOMS = 10000
N_FEAT = 128
N_RBF = 16
N_EDGES = 160000
CUTOFF = 5.0

NCORE = 2
NSUB = 16
NCH = 8
CPC = NCH // NCORE
EB = 64
NB = 158
EPT = NB * EB
E_PAD = EPT * NSUB
NPT = N_ATOMS // NSUB

_MLP_BLK = 1024
_NODE_BLK = 400


def _silu(x):
    return x * jax.nn.sigmoid(x)


def _mlp_body(rfv_ref, d_ref, w1_ref, b1_ref, w2_ref, b2_ref, w3_ref, b3_ref,
              out_ref):
    h = _silu(jnp.dot(rfv_ref[...], w1_ref[...],
                      preferred_element_type=jnp.float32) + b1_ref[...])
    h = _silu(jnp.dot(h, w2_ref[...],
                      preferred_element_type=jnp.float32) + b2_ref[...])
    h = _silu(jnp.dot(h, w3_ref[...],
                      preferred_element_type=jnp.float32) + b3_ref[...])
    d = d_ref[...]
    c = jnp.where(d < CUTOFF,
                  0.5 * (jnp.cos((jnp.pi / CUTOFF) * d) + 1.0), 0.0)
    out_ref[...] = h * c


def _mlp_call(rfv, d, w1, b1, w2, b2, w3p, b3p):
    grid = E_PAD // _MLP_BLK
    return pl.pallas_call(
        _mlp_body,
        grid=(grid,),
        in_specs=[
            pl.BlockSpec((_MLP_BLK, N_RBF), lambda i: (i, 0)),
            pl.BlockSpec((_MLP_BLK, 1), lambda i: (i, 0)),
            pl.BlockSpec((N_RBF, N_FEAT), lambda i: (0, 0)),
            pl.BlockSpec((1, N_FEAT), lambda i: (0, 0)),
            pl.BlockSpec((N_FEAT, 2 * N_FEAT), lambda i: (0, 0)),
            pl.BlockSpec((1, 2 * N_FEAT), lambda i: (0, 0)),
            pl.BlockSpec((2 * N_FEAT, 3 * N_FEAT), lambda i: (0, 0)),
            pl.BlockSpec((1, 3 * N_FEAT), lambda i: (0, 0)),
        ],
        out_specs=pl.BlockSpec((_MLP_BLK, 3 * N_FEAT), lambda i: (i, 0)),
        out_shape=jax.ShapeDtypeStruct((E_PAD, 3 * N_FEAT), jnp.float32),
        compiler_params=pltpu.CompilerParams(
            dimension_semantics=("parallel",)),
    )(rfv, d, w1, b1, w2, b2, w3p, b3p)


def _prep_body(x_ref, l0_ref, l1_ref, l2_ref, u_ref, xn_ref):
    xs = [x_ref[k] for k in range(9)]
    nrm = xs[0] * xs[0]
    for k in range(1, 9):
        nrm = nrm + xs[k] * xs[k]
    inv = 1.0 / (nrm + 1.0)
    xn = [v * inv for v in xs]
    for k in range(9):
        xn_ref[k] = xn[k]
    dg = (xn[0] + xn[4] + xn[8]) * (1.0 / 3.0)
    up = [dg,
          0.5 * (xn[1] - xn[3]), 0.5 * (xn[2] - xn[6]), 0.5 * (xn[5] - xn[7]),
          0.5 * (xn[1] + xn[3]), 0.5 * (xn[2] + xn[6]), 0.5 * (xn[5] + xn[7]),
          xn[0] - dg, xn[4] - dg]
    u_ref[0] = jnp.dot(up[0], l0_ref[...], preferred_element_type=jnp.float32)
    for k in range(1, 4):
        u_ref[k] = jnp.dot(up[k], l1_ref[...],
                           preferred_element_type=jnp.float32)
    for k in range(4, 9):
        u_ref[k] = jnp.dot(up[k], l2_ref[...],
                           preferred_element_type=jnp.float32)


def _prep_call(xr, l0, l1, l2):
    grid = N_ATOMS // _NODE_BLK
    lmat = lambda: pl.BlockSpec((N_FEAT, N_FEAT), lambda i: (0, 0))
    tens = lambda: pl.BlockSpec((9, _NODE_BLK, N_FEAT), lambda i: (0, i, 0))
    return pl.pallas_call(
        _prep_body,
        grid=(grid,),
        in_specs=[tens(), lmat(), lmat(), lmat()],
        out_specs=[tens(), tens()],
        out_shape=[jax.ShapeDtypeStruct((9, N_ATOMS, N_FEAT), jnp.float32),
                   jax.ShapeDtypeStruct((9, N_ATOMS, N_FEAT), jnp.float32)],
        compiler_params=pltpu.CompilerParams(
            dimension_semantics=("parallel",)),
    )(xr, l0, l1, l2)


def _sc_body(table_ref, w_ref, idx_ref, out_ref,
             ibuf0, ibuf1, gbuf0, gbuf1, wbuf0, wbuf1, zbuf, acc,
             isem0, isem1, gsem0, gsem1, wsem0, wsem1):
    c = lax.axis_index("c")
    s = lax.axis_index("s")
    e0 = s * EPT
    base = s * NPT

    def zrow(i, carry):
        for k in range(9):
            zbuf[i, k] = jnp.zeros((16,), jnp.float32)
        return carry
    lax.fori_loop(0, 25, zrow, 0)

    for cl in range(CPC):
        chunk = c * CPC + cl
        for z in range(25):
            pltpu.sync_copy(zbuf, acc.at[pl.ds(base + z * 25, 25)])
        plsc.subcore_barrier()

        def i_idx(b, ib, sem):
            pltpu.async_copy(idx_ref.at[chunk, s, b], ib, sem)

        def w_idx(b, ib, sem):
            pltpu.make_async_copy(idx_ref.at[chunk, s, b], ib, sem).wait()

        def i_gw(b, ib, gb, gsem, wb, wsem):
            pltpu.async_copy(table_ref.at[ib.at[0]], gb, gsem)
            pltpu.async_copy(w_ref.at[pl.ds(e0 + b * EB, EB), chunk], wb,
                             wsem)

        def w_gw(b, ib, gb, gsem, wb, wsem):
            pltpu.make_async_copy(table_ref.at[ib.at[0]], gb, gsem).wait()
            pltpu.make_async_copy(w_ref.at[pl.ds(e0 + b * EB, EB), chunk],
                                  wb, wsem).wait()

        def proc(gb, wb, ib):
            def edge(e4, cc):
                for u in range(4):
                    e = e4 * 4 + u
                    w0 = wb[e, pl.ds(0, 16)]
                    w1 = wb[e, pl.ds(16, 16)]
                    w2 = wb[e, pl.ds(32, 16)]
                    gb[e, 0] = gb[e, 0] * w0
                    for k in (1, 2, 3):
                        gb[e, k] = gb[e, k] * w1
                    for k in (4, 5, 6, 7, 8):
                        gb[e, k] = gb[e, k] * w2
                return cc
            lax.fori_loop(0, EB // 4, edge, 0)
            pltpu.sync_copy(gb, acc.at[ib.at[1]], add=True)

        i_idx(0, ibuf0, isem0)
        w_idx(0, ibuf0, isem0)
        i_gw(0, ibuf0, gbuf0, gsem0, wbuf0, wsem0)
        i_idx(1, ibuf1, isem1)

        def pair(t, carry):
            b0 = 2 * t
            w_idx(b0 + 1, ibuf1, isem1)
            i_gw(b0 + 1, ibuf1, gbuf1, gsem1, wbuf1, wsem1)
            w_gw(b0, ibuf0, gbuf0, gsem0, wbuf0, wsem0)
            proc(gbuf0, wbuf0, ibuf0)
            i_idx(b0 + 2, ibuf0, isem0)
            w_idx(b0 + 2, ibuf0, isem0)
            i_gw(b0 + 2, ibuf0, gbuf0, gsem0, wbuf0, wsem0)
            w_gw(b0 + 1, ibuf1, gbuf1, gsem1, wbuf1, wsem1)
            proc(gbuf1, wbuf1, ibuf1)
            i_idx(b0 + 3, ibuf1, isem1)
            return carry
        lax.fori_loop(0, NB // 2 - 1, pair, 0)

        w_idx(NB - 1, ibuf1, isem1)
        i_gw(NB - 1, ibuf1, gbuf1, gsem1, wbuf1, wsem1)
        w_gw(NB - 2, ibuf0, gbuf0, gsem0, wbuf0, wsem0)
        proc(gbuf0, wbuf0, ibuf0)
        w_gw(NB - 1, ibuf1, gbuf1, gsem1, wbuf1, wsem1)
        proc(gbuf1, wbuf1, ibuf1)

        plsc.subcore_barrier()
        pltpu.sync_copy(acc.at[pl.ds(base, NPT)],
                        out_ref.at[chunk, pl.ds(base, NPT)])
        plsc.subcore_barrier()


def _sc_call(table, w, idx_hbm):
    mesh = plsc.VectorSubcoreMesh(core_axis_name="c", subcore_axis_name="s")
    fn = pl.kernel(
        _sc_body,
        out_type=jax.ShapeDtypeStruct((NCH, N_ATOMS, 9, 16), jnp.float32),
        mesh=mesh,
        scratch_types=[
            pltpu.VMEM((2, EB), jnp.int32),
            pltpu.VMEM((2, EB), jnp.int32),
            pltpu.VMEM((EB, 9, 16), jnp.float32),
            pltpu.VMEM((EB, 9, 16), jnp.float32),
            pltpu.VMEM((EB, 48), jnp.float32),
            pltpu.VMEM((EB, 48), jnp.float32),
            pltpu.VMEM((25, 9, 16), jnp.float32),
            pltpu.VMEM_SHARED((N_ATOMS, 9, 16), jnp.float32),
            pltpu.SemaphoreType.DMA,
            pltpu.SemaphoreType.DMA,
            pltpu.SemaphoreType.DMA,
            pltpu.SemaphoreType.DMA,
            pltpu.SemaphoreType.DMA,
            pltpu.SemaphoreType.DMA,
        ],
        compiler_params=pltpu.CompilerParams(use_tc_tiling_on_sc=False),
    )
    return fn(table, w, idx_hbm)


def _decompress(u):
    return [u[0] + u[7], u[1] + u[4], u[2] + u[5],
            u[4] - u[1], u[0] + u[8], u[3] + u[6],
            u[5] - u[2], u[6] - u[3], u[0] - u[7] - u[8]]


def _combine_body(msg_ref, u_ref, xn_ref, q_ref, l3_ref, l4_ref, l5_ref,
                  out_ref):
    m = _decompress([msg_ref[k] for k in range(9)])
    y = _decompress([u_ref[k] for k in range(9)])
    t = [None] * 9
    for i in range(3):
        for k in range(3):
            acc = None
            for j in range(3):
                term = m[3 * i + j] * y[3 * j + k] + y[3 * i + j] * m[3 * j + k]
                acc = term if acc is None else acc + term
            t[3 * i + k] = acc
    nrm = t[0] * t[0]
    for k in range(1, 9):
        nrm = nrm + t[k] * t[k]
    inv = 1.0 / (nrm + 1.0)
    dg = (t[0] + t[4] + t[8]) * (1.0 / 3.0)
    v = [dg,
         0.5 * (t[1] - t[3]), 0.5 * (t[2] - t[6]), 0.5 * (t[5] - t[7]),
         0.5 * (t[1] + t[3]), 0.5 * (t[2] + t[6]), 0.5 * (t[5] + t[7]),
         t[0] - dg, t[4] - dg]
    v = [vi * inv for vi in v]
    w = [jnp.dot(v[0], l3_ref[...], preferred_element_type=jnp.float32)]
    for k in range(1, 4):
        w.append(jnp.dot(v[k], l4_ref[...],
                         preferred_element_type=jnp.float32))
    for k in range(4, 9):
        w.append(jnp.dot(v[k], l5_ref[...],
                         preferred_element_type=jnp.float32))
    dx = _decompress(w)
    f = 1.0 + 0.1 * q_ref[...]
    for i in range(3):
        for k in range(3):
            acc = None
            for j in range(3):
                term = dx[3 * i + j] * dx[3 * j + k]
                acc = term if acc is None else acc + term
            out_ref[3 * i + k] = (xn_ref[3 * i + k] + dx[3 * i + k]
                                  + f * acc)


def _combine_call(msg, u_lin, xn, q, l3, l4, l5):
    grid = N_ATOMS // _NODE_BLK
    lmat = lambda: pl.BlockSpec((N_FEAT, N_FEAT), lambda i: (0, 0))
    tens = lambda: pl.BlockSpec((9, _NODE_BLK, N_FEAT), lambda i: (0, i, 0))
    return pl.pallas_call(
        _combine_body,
        grid=(grid,),
        in_specs=[tens(), tens(), tens(),
                  pl.BlockSpec((_NODE_BLK, 1), lambda i: (i, 0)),
                  lmat(), lmat(), lmat()],
        out_specs=tens(),
        out_shape=jax.ShapeDtypeStruct((9, N_ATOMS, N_FEAT), jnp.float32),
        compiler_params=pltpu.CompilerParams(
            dimension_semantics=("parallel",)),
    )(msg, u_lin, xn, q, l3, l4, l5)


def kernel(X, pair_indices, d_ij, radial_feature_vector, atomic_charges,
           W1, b1, W2, b2, W3, b3, L0, L1, L2, L3, L4, L5):
    cols = jnp.arange(3 * N_FEAT)
    cchunk = cols // 48
    rem = cols % 48
    chan = rem // 16
    feat = rem % 16
    src_col = (cchunk * 16 + feat) * 3 + chan
    w3p = W3[:, src_col]
    b3p = b3[src_col]

    pad = E_PAD - N_EDGES
    rfv_p = jnp.pad(radial_feature_vector, ((0, pad), (0, 0)))
    d_p = jnp.pad(d_ij, ((0, pad), (0, 0)), constant_values=CUTOFF)
    pi = pair_indices.astype(jnp.int32)
    dst = jnp.pad(pi[0], (0, pad))
    src = jnp.pad(pi[1], (0, pad))
    src_shift = src[None, :] + (jnp.arange(NCH, dtype=jnp.int32)
                                * N_ATOMS)[:, None]
    src_r = src_shift.reshape(NCH, NSUB, NB, 1, EB)
    dst_r = jnp.broadcast_to(dst.reshape(1, NSUB, NB, 1, EB),
                             (NCH, NSUB, NB, 1, EB))
    idx_hbm = jnp.concatenate([src_r, dst_r], axis=3)

    w = _mlp_call(rfv_p, d_p, W1, b1.reshape(1, -1), W2, b2.reshape(1, -1),
                  w3p, b3p.reshape(1, -1))
    w = w.reshape(E_PAD, NCH, 48)

    xr = X.transpose(2, 3, 0, 1).reshape(9, N_ATOMS, N_FEAT)
    u_lin, xn = _prep_call(xr, L0, L1, L2)

    table = (u_lin.reshape(9, N_ATOMS, NCH, 16)
             .transpose(2, 1, 0, 3)
             .reshape(NCH * N_ATOMS, 9, 16))
    msg_chunks = _sc_call(table, w, idx_hbm)
    msg = (msg_chunks.transpose(2, 1, 0, 3)
           .reshape(9, N_ATOMS, N_FEAT))

    xout = _combine_call(msg, u_lin, xn, atomic_charges.reshape(N_ATOMS, 1),
                         L3, L4, L5)
    return xout.reshape(3, 3, N_ATOMS, N_FEAT).transpose(2, 3, 0, 1)


# speedup vs baseline: 19.7731x; 1.0243x over previous
import functools

import jax
import jax.numpy as jnp
import numpy as np
from jax import lax
from jax.experimental import pallas as pl
from jax.experimental.pallas import tpu as pltpu
from jax.experimental.pallas import tpu_sc as plsc

N_ATOMS = 10000
N_FEAT = 128
N_RBF = 16
N_EDGES = 160000
CUTOFF = 5.0

NCORE = 2
NSUB = 16
NCH = 8
CPC = NCH // NCORE
EB = 64
NB = 158
EPT = NB * EB
E_PAD = EPT * NSUB
NPT = N_ATOMS // NSUB

_MLP_BLK = 1024
_NODE_BLK = 400

_PX = np.zeros((1152, 1152), np.float32)
for _f in range(128):
    for _k in range(9):
        _PX[_f * 9 + _k, _k * 128 + _f] = 1.0
_PT = np.zeros((1152, 1152), np.float32)
for _c in range(8):
    for _k in range(9):
        for _f in range(16):
            _PT[_k * 128 + _c * 16 + _f, _c * 144 + _k * 16 + _f] = 1.0
_PM = _PT.T.copy()
_PO = _PX.T.copy()


def _silu(x):
    return x * jax.nn.sigmoid(x)


def _mlp_body(rfv_ref, d_ref, w1_ref, b1_ref, w2_ref, b2_ref, w3_ref, b3_ref,
              out_ref):
    h = _silu(jnp.dot(rfv_ref[...], w1_ref[...],
                      preferred_element_type=jnp.float32) + b1_ref[...])
    h = _silu(jnp.dot(h, w2_ref[...],
                      preferred_element_type=jnp.float32) + b2_ref[...])
    h = _silu(jnp.dot(h, w3_ref[...],
                      preferred_element_type=jnp.float32) + b3_ref[...])
    d = d_ref[...]
    c = jnp.where(d < CUTOFF,
                  0.5 * (jnp.cos((jnp.pi / CUTOFF) * d) + 1.0), 0.0)
    out_ref[...] = h * c


def _mlp_call(rfv, d, w1, b1, w2, b2, w3p, b3p):
    grid = E_PAD // _MLP_BLK
    return pl.pallas_call(
        _mlp_body,
        grid=(grid,),
        in_specs=[
            pl.BlockSpec((_MLP_BLK, N_RBF), lambda i: (i, 0)),
            pl.BlockSpec((_MLP_BLK, 1), lambda i: (i, 0)),
            pl.BlockSpec((N_RBF, N_FEAT), lambda i: (0, 0)),
            pl.BlockSpec((1, N_FEAT), lambda i: (0, 0)),
            pl.BlockSpec((N_FEAT, 2 * N_FEAT), lambda i: (0, 0)),
            pl.BlockSpec((1, 2 * N_FEAT), lambda i: (0, 0)),
            pl.BlockSpec((2 * N_FEAT, 3 * N_FEAT), lambda i: (0, 0)),
            pl.BlockSpec((1, 3 * N_FEAT), lambda i: (0, 0)),
        ],
        out_specs=pl.BlockSpec((_MLP_BLK, 3 * N_FEAT), lambda i: (i, 0)),
        out_shape=jax.ShapeDtypeStruct((E_PAD, 3 * N_FEAT), jnp.float32),
        compiler_params=pltpu.CompilerParams(
            dimension_semantics=("parallel",)),
    )(rfv, d, w1, b1, w2, b2, w3p, b3p)


def _prep_body(x_ref, px_ref, pt_ref, l0_ref, l1_ref, l2_ref,
               uperm_ref, u_ref, xn_ref):
    xc = jnp.dot(x_ref[...], px_ref[...], preferred_element_type=jnp.float32)
    xs = [xc[:, k * N_FEAT:(k + 1) * N_FEAT] for k in range(9)]
    nrm = xs[0] * xs[0]
    for k in range(1, 9):
        nrm = nrm + xs[k] * xs[k]
    inv = 1.0 / (nrm + 1.0)
    xn = [v * inv for v in xs]
    for k in range(9):
        xn_ref[k] = xn[k]
    dg = (xn[0] + xn[4] + xn[8]) * (1.0 / 3.0)
    up = [dg,
          0.5 * (xn[1] - xn[3]), 0.5 * (xn[2] - xn[6]), 0.5 * (xn[5] - xn[7]),
          0.5 * (xn[1] + xn[3]), 0.5 * (xn[2] + xn[6]), 0.5 * (xn[5] + xn[7]),
          xn[0] - dg, xn[4] - dg]
    u = [jnp.dot(up[0], l0_ref[...], preferred_element_type=jnp.float32)]
    for k in range(1, 4):
        u.append(jnp.dot(up[k], l1_ref[...],
                         preferred_element_type=jnp.float32))
    for k in range(4, 9):
        u.append(jnp.dot(up[k], l2_ref[...],
                         preferred_element_type=jnp.float32))
    for k in range(9):
        u_ref[k] = u[k]
    ucat = jnp.concatenate(u, axis=1)
    uperm_ref[...] = jnp.dot(ucat, pt_ref[...],
                             preferred_element_type=jnp.float32)


def _prep_call(x2, l0, l1, l2):
    grid = N_ATOMS // _NODE_BLK
    lmat = lambda: pl.BlockSpec((N_FEAT, N_FEAT), lambda i: (0, 0))
    pmat = lambda: pl.BlockSpec((1152, 1152), lambda i: (0, 0))
    wide = lambda: pl.BlockSpec((_NODE_BLK, 1152), lambda i: (i, 0))
    tens = lambda: pl.BlockSpec((9, _NODE_BLK, N_FEAT), lambda i: (0, i, 0))
    return pl.pallas_call(
        _prep_body,
        grid=(grid,),
        in_specs=[wide(), pmat(), pmat(), lmat(), lmat(), lmat()],
        out_specs=[wide(), tens(), tens()],
        out_shape=[jax.ShapeDtypeStruct((N_ATOMS, 1152), jnp.float32),
                   jax.ShapeDtypeStruct((9, N_ATOMS, N_FEAT), jnp.float32),
                   jax.ShapeDtypeStruct((9, N_ATOMS, N_FEAT), jnp.float32)],
        compiler_params=pltpu.CompilerParams(
            dimension_semantics=("parallel",)),
    )(x2, jnp.asarray(_PX), jnp.asarray(_PT), l0, l1, l2)


def _sc_body(table_ref, w_ref, idx_ref, out_ref,
             ibuf0, ibuf1, gbuf0, gbuf1, wbuf0, wbuf1, zbuf, acc,
             isem0, isem1, gsem0, gsem1, wsem0, wsem1):
    c = lax.axis_index("c")
    s = lax.axis_index("s")
    e0 = s * EPT
    base = s * NPT

    def zrow(i, carry):
        for k in range(9):
            zbuf[i, k] = jnp.zeros((16,), jnp.float32)
        return carry
    lax.fori_loop(0, 25, zrow, 0)

    for cl in range(CPC):
        chunk = c * CPC + cl
        for z in range(25):
            pltpu.sync_copy(zbuf, acc.at[pl.ds(base + z * 25, 25)])
        plsc.subcore_barrier()

        def i_idx(b, ib, sem):
            pltpu.async_copy(idx_ref.at[chunk, s, b], ib, sem)

        def w_idx(b, ib, sem):
            pltpu.make_async_copy(idx_ref.at[chunk, s, b], ib, sem).wait()

        def i_gw(b, ib, gb, gsem, wb, wsem):
            pltpu.async_copy(table_ref.at[ib.at[0]], gb, gsem)
            pltpu.async_copy(w_ref.at[pl.ds(e0 + b * EB, EB), chunk], wb,
                             wsem)

        def w_gw(b, ib, gb, gsem, wb, wsem):
            pltpu.make_async_copy(table_ref.at[ib.at[0]], gb, gsem).wait()
            pltpu.make_async_copy(w_ref.at[pl.ds(e0 + b * EB, EB), chunk],
                                  wb, wsem).wait()

        def proc(gb, wb, ib):
            def edge(e4, cc):
                for u in range(4):
                    e = e4 * 4 + u
                    w0 = wb[e, pl.ds(0, 16)]
                    w1 = wb[e, pl.ds(16, 16)]
                    w2 = wb[e, pl.ds(32, 16)]
                    gb[e, 0] = gb[e, 0] * w0
                    for k in (1, 2, 3):
                        gb[e, k] = gb[e, k] * w1
                    for k in (4, 5, 6, 7, 8):
                        gb[e, k] = gb[e, k] * w2
                return cc
            lax.fori_loop(0, EB // 4, edge, 0)
            pltpu.sync_copy(gb, acc.at[ib.at[1]], add=True)

        i_idx(0, ibuf0, isem0)
        w_idx(0, ibuf0, isem0)
        i_gw(0, ibuf0, gbuf0, gsem0, wbuf0, wsem0)
        i_idx(1, ibuf1, isem1)

        def pair(t, carry):
            b0 = 2 * t
            w_idx(b0 + 1, ibuf1, isem1)
            i_gw(b0 + 1, ibuf1, gbuf1, gsem1, wbuf1, wsem1)
            w_gw(b0, ibuf0, gbuf0, gsem0, wbuf0, wsem0)
            proc(gbuf0, wbuf0, ibuf0)
            i_idx(b0 + 2, ibuf0, isem0)
            w_idx(b0 + 2, ibuf0, isem0)
            i_gw(b0 + 2, ibuf0, gbuf0, gsem0, wbuf0, wsem0)
            w_gw(b0 + 1, ibuf1, gbuf1, gsem1, wbuf1, wsem1)
            proc(gbuf1, wbuf1, ibuf1)
            i_idx(b0 + 3, ibuf1, isem1)
            return carry
        lax.fori_loop(0, NB // 2 - 1, pair, 0)

        w_idx(NB - 1, ibuf1, isem1)
        i_gw(NB - 1, ibuf1, gbuf1, gsem1, wbuf1, wsem1)
        w_gw(NB - 2, ibuf0, gbuf0, gsem0, wbuf0, wsem0)
        proc(gbuf0, wbuf0, ibuf0)
        w_gw(NB - 1, ibuf1, gbuf1, gsem1, wbuf1, wsem1)
        proc(gbuf1, wbuf1, ibuf1)

        plsc.subcore_barrier()
        pltpu.sync_copy(acc.at[pl.ds(base, NPT)],
                        out_ref.at[pl.ds(base, NPT), chunk])
        plsc.subcore_barrier()


def _sc_call(table, w, idx_hbm):
    mesh = plsc.VectorSubcoreMesh(core_axis_name="c", subcore_axis_name="s")
    fn = pl.kernel(
        _sc_body,
        out_type=jax.ShapeDtypeStruct((N_ATOMS, NCH, 9, 16), jnp.float32),
        mesh=mesh,
        scratch_types=[
            pltpu.VMEM((2, EB), jnp.int32),
            pltpu.VMEM((2, EB), jnp.int32),
            pltpu.VMEM((EB, 9, 16), jnp.float32),
            pltpu.VMEM((EB, 9, 16), jnp.float32),
            pltpu.VMEM((EB, 48), jnp.float32),
            pltpu.VMEM((EB, 48), jnp.float32),
            pltpu.VMEM((25, 9, 16), jnp.float32),
            pltpu.VMEM_SHARED((N_ATOMS, 9, 16), jnp.float32),
            pltpu.SemaphoreType.DMA,
            pltpu.SemaphoreType.DMA,
            pltpu.SemaphoreType.DMA,
            pltpu.SemaphoreType.DMA,
            pltpu.SemaphoreType.DMA,
            pltpu.SemaphoreType.DMA,
        ],
        compiler_params=pltpu.CompilerParams(use_tc_tiling_on_sc=False),
    )
    return fn(table, w, idx_hbm)


def _decompress(u):
    return [u[0] + u[7], u[1] + u[4], u[2] + u[5],
            u[4] - u[1], u[0] + u[8], u[3] + u[6],
            u[5] - u[2], u[6] - u[3], u[0] - u[7] - u[8]]


def _combine_body(msg_ref, u_ref, xn_ref, q_ref, pm_ref, l3_ref, l4_ref,
                  l5_ref, po_ref, out_ref):
    mc = jnp.dot(msg_ref[...], pm_ref[...],
                 preferred_element_type=jnp.float32)
    m = _decompress([mc[:, k * N_FEAT:(k + 1) * N_FEAT] for k in range(9)])
    y = _decompress([u_ref[k] for k in range(9)])
    t = [None] * 9
    for i in range(3):
        for k in range(3):
            acc = None
            for j in range(3):
                term = m[3 * i + j] * y[3 * j + k] + y[3 * i + j] * m[3 * j + k]
                acc = term if acc is None else acc + term
            t[3 * i + k] = acc
    nrm = t[0] * t[0]
    for k in range(1, 9):
        nrm = nrm + t[k] * t[k]
    inv = 1.0 / (nrm + 1.0)
    dg = (t[0] + t[4] + t[8]) * (1.0 / 3.0)
    v = [dg,
         0.5 * (t[1] - t[3]), 0.5 * (t[2] - t[6]), 0.5 * (t[5] - t[7]),
         0.5 * (t[1] + t[3]), 0.5 * (t[2] + t[6]), 0.5 * (t[5] + t[7]),
         t[0] - dg, t[4] - dg]
    v = [vi * inv for vi in v]
    w = [jnp.dot(v[0], l3_ref[...], preferred_element_type=jnp.float32)]
    for k in range(1, 4):
        w.append(jnp.dot(v[k], l4_ref[...],
                         preferred_element_type=jnp.float32))
    for k in range(4, 9):
        w.append(jnp.dot(v[k], l5_ref[...],
                         preferred_element_type=jnp.float32))
    dx = _decompress(w)
    f = 1.0 + 0.1 * q_ref[...]
    outc = [None] * 9
    for i in range(3):
        for k in range(3):
            acc = None
            for j in range(3):
                term = dx[3 * i + j] * dx[3 * j + k]
                acc = term if acc is None else acc + term
            outc[3 * i + k] = (xn_ref[3 * i + k] + dx[3 * i + k] + f * acc)
    ocat = jnp.concatenate(outc, axis=1)
    out_ref[...] = jnp.dot(ocat, po_ref[...],
                           preferred_element_type=jnp.float32)


def _combine_call(msg_perm, u_lin, xn, q, l3, l4, l5):
    grid = N_ATOMS // _NODE_BLK
    lmat = lambda: pl.BlockSpec((N_FEAT, N_FEAT), lambda i: (0, 0))
    pmat = lambda: pl.BlockSpec((1152, 1152), lambda i: (0, 0))
    wide = lambda: pl.BlockSpec((_NODE_BLK, 1152), lambda i: (i, 0))
    tens = lambda: pl.BlockSpec((9, _NODE_BLK, N_FEAT), lambda i: (0, i, 0))
    return pl.pallas_call(
        _combine_body,
        grid=(grid,),
        in_specs=[wide(), tens(), tens(),
                  pl.BlockSpec((_NODE_BLK, 1), lambda i: (i, 0)),
                  pmat(), lmat(), lmat(), lmat(), pmat()],
        out_specs=wide(),
        out_shape=jax.ShapeDtypeStruct((N_ATOMS, 1152), jnp.float32),
        compiler_params=pltpu.CompilerParams(
            dimension_semantics=("parallel",)),
    )(msg_perm, u_lin, xn, q, jnp.asarray(_PM), l3, l4, l5,
      jnp.asarray(_PO))


def kernel(X, pair_indices, d_ij, radial_feature_vector, atomic_charges,
           W1, b1, W2, b2, W3, b3, L0, L1, L2, L3, L4, L5):
    cols = jnp.arange(3 * N_FEAT)
    cchunk = cols // 48
    rem = cols % 48
    chan = rem // 16
    feat = rem % 16
    src_col = (cchunk * 16 + feat) * 3 + chan
    w3p = W3[:, src_col]
    b3p = b3[src_col]

    pad = E_PAD - N_EDGES
    rfv_p = jnp.pad(radial_feature_vector, ((0, pad), (0, 0)))
    d_p = jnp.pad(d_ij, ((0, pad), (0, 0)), constant_values=CUTOFF)
    pi = pair_indices.astype(jnp.int32)
    dst = jnp.pad(pi[0], (0, pad))
    src = jnp.pad(pi[1], (0, pad))
    src_shift = src[None, :] * NCH + jnp.arange(
        NCH, dtype=jnp.int32)[:, None]
    src_r = src_shift.reshape(NCH, NSUB, NB, 1, EB)
    dst_r = jnp.broadcast_to(dst.reshape(1, NSUB, NB, 1, EB),
                             (NCH, NSUB, NB, 1, EB))
    idx_hbm = jnp.concatenate([src_r, dst_r], axis=3)

    w = _mlp_call(rfv_p, d_p, W1, b1.reshape(1, -1), W2, b2.reshape(1, -1),
                  w3p, b3p.reshape(1, -1))
    w = w.reshape(E_PAD, NCH, 48)

    x2 = X.reshape(N_ATOMS, 1152)
    u_perm, u_lin, xn = _prep_call(x2, L0, L1, L2)

    table = u_perm.reshape(NCH * N_ATOMS, 9, 16)
    msg_chunks = _sc_call(table, w, idx_hbm)
    msg_perm = msg_chunks.reshape(N_ATOMS, 1152)

    xout = _combine_call(msg_perm, u_lin, xn,
                         atomic_charges.reshape(N_ATOMS, 1), L3, L4, L5)
    return xout.reshape(N_ATOMS, N_FEAT, 3, 3)


# speedup vs baseline: 23.8574x; 1.2066x over previous
import functools

import jax
import jax.numpy as jnp
import numpy as np
from jax import lax
from jax.experimental import pallas as pl
from jax.experimental.pallas import tpu as pltpu
from jax.experimental.pallas import tpu_sc as plsc

N_ATOMS = 10000
N_FEAT = 128
N_RBF = 16
N_EDGES = 160000
CUTOFF = 5.0

NCORE = 2
NSUB = 16
NCH = 8
CPC = NCH // NCORE
EB = 64
NB = 158
EPT = NB * EB
E_PAD = EPT * NSUB
NPT = N_ATOMS // NSUB

_MLP_BLK = 1024
_NODE_BLK = 400

_PX = np.zeros((1152, 1152), np.float32)
for _f in range(128):
    for _k in range(9):
        _PX[_f * 9 + _k, _k * 128 + _f] = 1.0
_PT = np.zeros((1152, 1152), np.float32)
for _c in range(8):
    for _k in range(9):
        for _f in range(16):
            _PT[_k * 128 + _c * 16 + _f, _c * 144 + _k * 16 + _f] = 1.0
_PM = _PT.T.copy()
_PO = _PX.T.copy()


def _silu(x):
    return x * jax.nn.sigmoid(x)


def _mlp_body(rfv_ref, d_ref, w1_ref, b1_ref, w2_ref, b2_ref, w3_ref, b3_ref,
              out_ref):
    h = _silu(jnp.dot(rfv_ref[...], w1_ref[...],
                      preferred_element_type=jnp.float32) + b1_ref[...])
    h = _silu(jnp.dot(h, w2_ref[...],
                      preferred_element_type=jnp.float32) + b2_ref[...])
    h = _silu(jnp.dot(h, w3_ref[...],
                      preferred_element_type=jnp.float32) + b3_ref[...])
    d = d_ref[...]
    c = jnp.where(d < CUTOFF,
                  0.5 * (jnp.cos((jnp.pi / CUTOFF) * d) + 1.0), 0.0)
    out_ref[...] = h * c


def _mlp_call(rfv, d, w1, b1, w2, b2, w3p, b3p):
    grid = E_PAD // _MLP_BLK
    return pl.pallas_call(
        _mlp_body,
        grid=(grid,),
        in_specs=[
            pl.BlockSpec((_MLP_BLK, N_RBF), lambda i: (i, 0)),
            pl.BlockSpec((_MLP_BLK, 1), lambda i: (i, 0)),
            pl.BlockSpec((N_RBF, N_FEAT), lambda i: (0, 0)),
            pl.BlockSpec((1, N_FEAT), lambda i: (0, 0)),
            pl.BlockSpec((N_FEAT, 2 * N_FEAT), lambda i: (0, 0)),
            pl.BlockSpec((1, 2 * N_FEAT), lambda i: (0, 0)),
            pl.BlockSpec((2 * N_FEAT, 3 * N_FEAT), lambda i: (0, 0)),
            pl.BlockSpec((1, 3 * N_FEAT), lambda i: (0, 0)),
        ],
        out_specs=pl.BlockSpec((_MLP_BLK, 3 * N_FEAT), lambda i: (i, 0)),
        out_shape=jax.ShapeDtypeStruct((E_PAD, 3 * N_FEAT), jnp.float32),
        compiler_params=pltpu.CompilerParams(
            dimension_semantics=("parallel",)),
    )(rfv, d, w1, b1, w2, b2, w3p, b3p)


def _prep_body(x_ref, px_ref, pt_ref, l0_ref, l1_ref, l2_ref,
               uperm_ref, u_ref, xn_ref):
    xc = jnp.dot(x_ref[...], px_ref[...], preferred_element_type=jnp.float32)
    xs = [xc[:, k * N_FEAT:(k + 1) * N_FEAT] for k in range(9)]
    nrm = xs[0] * xs[0]
    for k in range(1, 9):
        nrm = nrm + xs[k] * xs[k]
    inv = 1.0 / (nrm + 1.0)
    xn = [v * inv for v in xs]
    for k in range(9):
        xn_ref[k] = xn[k]
    dg = (xn[0] + xn[4] + xn[8]) * (1.0 / 3.0)
    up = [dg,
          0.5 * (xn[1] - xn[3]), 0.5 * (xn[2] - xn[6]), 0.5 * (xn[5] - xn[7]),
          0.5 * (xn[1] + xn[3]), 0.5 * (xn[2] + xn[6]), 0.5 * (xn[5] + xn[7]),
          xn[0] - dg, xn[4] - dg]
    u = [jnp.dot(up[0], l0_ref[...], preferred_element_type=jnp.float32)]
    for k in range(1, 4):
        u.append(jnp.dot(up[k], l1_ref[...],
                         preferred_element_type=jnp.float32))
    for k in range(4, 9):
        u.append(jnp.dot(up[k], l2_ref[...],
                         preferred_element_type=jnp.float32))
    for k in range(9):
        u_ref[k] = u[k]
    ucat = jnp.concatenate(u, axis=1)
    uperm_ref[...] = jnp.dot(ucat, pt_ref[...],
                             preferred_element_type=jnp.float32)


def _prep_call(x2, l0, l1, l2):
    grid = N_ATOMS // _NODE_BLK
    lmat = lambda: pl.BlockSpec((N_FEAT, N_FEAT), lambda i: (0, 0))
    pmat = lambda: pl.BlockSpec((1152, 1152), lambda i: (0, 0))
    wide = lambda: pl.BlockSpec((_NODE_BLK, 1152), lambda i: (i, 0))
    tens = lambda: pl.BlockSpec((9, _NODE_BLK, N_FEAT), lambda i: (0, i, 0))
    return pl.pallas_call(
        _prep_body,
        grid=(grid,),
        in_specs=[wide(), pmat(), pmat(), lmat(), lmat(), lmat()],
        out_specs=[wide(), tens(), tens()],
        out_shape=[jax.ShapeDtypeStruct((N_ATOMS, 1152), jnp.float32),
                   jax.ShapeDtypeStruct((9, N_ATOMS, N_FEAT), jnp.float32),
                   jax.ShapeDtypeStruct((9, N_ATOMS, N_FEAT), jnp.float32)],
        compiler_params=pltpu.CompilerParams(
            dimension_semantics=("parallel",)),
    )(x2, jnp.asarray(_PX), jnp.asarray(_PT), l0, l1, l2)


def _sc_body(table_ref, w_ref, idx_ref, out_ref,
             ibuf0, ibuf1, gbuf0, gbuf1, wbuf0, wbuf1, zbuf, acc,
             isem0, isem1, gsem0, gsem1, wsem0, wsem1):
    c = lax.axis_index("c")
    s = lax.axis_index("s")
    e0 = s * EPT
    base = s * NPT

    def zrow(i, carry):
        for k in range(9):
            zbuf[i, k] = jnp.zeros((16,), jnp.float32)
        return carry
    lax.fori_loop(0, 25, zrow, 0)

    for cl in range(CPC):
        chunk = c * CPC + cl
        for z in range(25):
            pltpu.sync_copy(zbuf, acc.at[pl.ds(base + z * 25, 25)])
        plsc.subcore_barrier()

        def i_idx(b, ib, sem):
            pltpu.async_copy(idx_ref.at[chunk, s, b], ib, sem)

        def w_idx(b, ib, sem):
            pltpu.make_async_copy(idx_ref.at[chunk, s, b], ib, sem).wait()

        def i_gw(b, ib, gb, gsem, wb, wsem):
            pltpu.async_copy(table_ref.at[ib.at[0]], gb, gsem)
            pltpu.async_copy(
                w_ref.at[pl.ds(e0 + b * EB, EB), pl.ds(chunk * 48, 48)],
                wb, wsem)

        def w_gw(b, ib, gb, gsem, wb, wsem):
            pltpu.make_async_copy(table_ref.at[ib.at[0]], gb, gsem).wait()
            pltpu.make_async_copy(
                w_ref.at[pl.ds(e0 + b * EB, EB), pl.ds(chunk * 48, 48)],
                wb, wsem).wait()

        def proc(gb, wb, ib):
            def edge(e4, cc):
                for u in range(4):
                    e = e4 * 4 + u
                    w0 = wb[e, pl.ds(0, 16)]
                    w1 = wb[e, pl.ds(16, 16)]
                    w2 = wb[e, pl.ds(32, 16)]
                    gb[e, 0] = gb[e, 0] * w0
                    for k in (1, 2, 3):
                        gb[e, k] = gb[e, k] * w1
                    for k in (4, 5, 6, 7, 8):
                        gb[e, k] = gb[e, k] * w2
                return cc
            lax.fori_loop(0, EB // 4, edge, 0)
            pltpu.sync_copy(gb, acc.at[ib.at[1]], add=True)

        i_idx(0, ibuf0, isem0)
        w_idx(0, ibuf0, isem0)
        i_gw(0, ibuf0, gbuf0, gsem0, wbuf0, wsem0)
        i_idx(1, ibuf1, isem1)

        def pair(t, carry):
            b0 = 2 * t
            w_idx(b0 + 1, ibuf1, isem1)
            i_gw(b0 + 1, ibuf1, gbuf1, gsem1, wbuf1, wsem1)
            w_gw(b0, ibuf0, gbuf0, gsem0, wbuf0, wsem0)
            proc(gbuf0, wbuf0, ibuf0)
            i_idx(b0 + 2, ibuf0, isem0)
            w_idx(b0 + 2, ibuf0, isem0)
            i_gw(b0 + 2, ibuf0, gbuf0, gsem0, wbuf0, wsem0)
            w_gw(b0 + 1, ibuf1, gbuf1, gsem1, wbuf1, wsem1)
            proc(gbuf1, wbuf1, ibuf1)
            i_idx(b0 + 3, ibuf1, isem1)
            return carry
        lax.fori_loop(0, NB // 2 - 1, pair, 0)

        w_idx(NB - 1, ibuf1, isem1)
        i_gw(NB - 1, ibuf1, gbuf1, gsem1, wbuf1, wsem1)
        w_gw(NB - 2, ibuf0, gbuf0, gsem0, wbuf0, wsem0)
        proc(gbuf0, wbuf0, ibuf0)
        w_gw(NB - 1, ibuf1, gbuf1, gsem1, wbuf1, wsem1)
        proc(gbuf1, wbuf1, ibuf1)

        plsc.subcore_barrier()
        pltpu.sync_copy(acc.at[pl.ds(base, NPT)],
                        out_ref.at[pl.ds(base, NPT), chunk])
        plsc.subcore_barrier()


def _sc_call(table, w, idx_hbm):
    mesh = plsc.VectorSubcoreMesh(core_axis_name="c", subcore_axis_name="s")
    fn = pl.kernel(
        _sc_body,
        out_type=jax.ShapeDtypeStruct((N_ATOMS, NCH, 9, 16), jnp.float32),
        mesh=mesh,
        scratch_types=[
            pltpu.VMEM((2, EB), jnp.int32),
            pltpu.VMEM((2, EB), jnp.int32),
            pltpu.VMEM((EB, 9, 16), jnp.float32),
            pltpu.VMEM((EB, 9, 16), jnp.float32),
            pltpu.VMEM((EB, 48), jnp.float32),
            pltpu.VMEM((EB, 48), jnp.float32),
            pltpu.VMEM((25, 9, 16), jnp.float32),
            pltpu.VMEM_SHARED((N_ATOMS, 9, 16), jnp.float32),
            pltpu.SemaphoreType.DMA,
            pltpu.SemaphoreType.DMA,
            pltpu.SemaphoreType.DMA,
            pltpu.SemaphoreType.DMA,
            pltpu.SemaphoreType.DMA,
            pltpu.SemaphoreType.DMA,
        ],
        compiler_params=pltpu.CompilerParams(use_tc_tiling_on_sc=False),
    )
    return fn(table, w, idx_hbm)


def _decompress(u):
    return [u[0] + u[7], u[1] + u[4], u[2] + u[5],
            u[4] - u[1], u[0] + u[8], u[3] + u[6],
            u[5] - u[2], u[6] - u[3], u[0] - u[7] - u[8]]


def _combine_body(msg_ref, u_ref, xn_ref, q_ref, pm_ref, l3_ref, l4_ref,
                  l5_ref, po_ref, out_ref):
    mc = jnp.dot(msg_ref[...], pm_ref[...],
                 preferred_element_type=jnp.float32)
    m = _decompress([mc[:, k * N_FEAT:(k + 1) * N_FEAT] for k in range(9)])
    y = _decompress([u_ref[k] for k in range(9)])
    t = [None] * 9
    for i in range(3):
        for k in range(3):
            acc = None
            for j in range(3):
                term = m[3 * i + j] * y[3 * j + k] + y[3 * i + j] * m[3 * j + k]
                acc = term if acc is None else acc + term
            t[3 * i + k] = acc
    nrm = t[0] * t[0]
    for k in range(1, 9):
        nrm = nrm + t[k] * t[k]
    inv = 1.0 / (nrm + 1.0)
    dg = (t[0] + t[4] + t[8]) * (1.0 / 3.0)
    v = [dg,
         0.5 * (t[1] - t[3]), 0.5 * (t[2] - t[6]), 0.5 * (t[5] - t[7]),
         0.5 * (t[1] + t[3]), 0.5 * (t[2] + t[6]), 0.5 * (t[5] + t[7]),
         t[0] - dg, t[4] - dg]
    v = [vi * inv for vi in v]
    w = [jnp.dot(v[0], l3_ref[...], preferred_element_type=jnp.float32)]
    for k in range(1, 4):
        w.append(jnp.dot(v[k], l4_ref[...],
                         preferred_element_type=jnp.float32))
    for k in range(4, 9):
        w.append(jnp.dot(v[k], l5_ref[...],
                         preferred_element_type=jnp.float32))
    dx = _decompress(w)
    f = 1.0 + 0.1 * q_ref[...]
    outc = [None] * 9
    for i in range(3):
        for k in range(3):
            acc = None
            for j in range(3):
                term = dx[3 * i + j] * dx[3 * j + k]
                acc = term if acc is None else acc + term
            outc[3 * i + k] = (xn_ref[3 * i + k] + dx[3 * i + k] + f * acc)
    ocat = jnp.concatenate(outc, axis=1)
    out_ref[...] = jnp.dot(ocat, po_ref[...],
                           preferred_element_type=jnp.float32)


def _combine_call(msg_perm, u_lin, xn, q, l3, l4, l5):
    grid = N_ATOMS // _NODE_BLK
    lmat = lambda: pl.BlockSpec((N_FEAT, N_FEAT), lambda i: (0, 0))
    pmat = lambda: pl.BlockSpec((1152, 1152), lambda i: (0, 0))
    wide = lambda: pl.BlockSpec((_NODE_BLK, 1152), lambda i: (i, 0))
    tens = lambda: pl.BlockSpec((9, _NODE_BLK, N_FEAT), lambda i: (0, i, 0))
    return pl.pallas_call(
        _combine_body,
        grid=(grid,),
        in_specs=[wide(), tens(), tens(),
                  pl.BlockSpec((_NODE_BLK, 1), lambda i: (i, 0)),
                  pmat(), lmat(), lmat(), lmat(), pmat()],
        out_specs=wide(),
        out_shape=jax.ShapeDtypeStruct((N_ATOMS, 1152), jnp.float32),
        compiler_params=pltpu.CompilerParams(
            dimension_semantics=("parallel",)),
    )(msg_perm, u_lin, xn, q, jnp.asarray(_PM), l3, l4, l5,
      jnp.asarray(_PO))


def kernel(X, pair_indices, d_ij, radial_feature_vector, atomic_charges,
           W1, b1, W2, b2, W3, b3, L0, L1, L2, L3, L4, L5):
    cols = jnp.arange(3 * N_FEAT)
    cchunk = cols // 48
    rem = cols % 48
    chan = rem // 16
    feat = rem % 16
    src_col = (cchunk * 16 + feat) * 3 + chan
    w3p = W3[:, src_col]
    b3p = b3[src_col]

    pad = E_PAD - N_EDGES
    rfv_p = jnp.pad(radial_feature_vector, ((0, pad), (0, 0)))
    d_p = jnp.pad(d_ij, ((0, pad), (0, 0)), constant_values=CUTOFF)
    pi = pair_indices.astype(jnp.int32)
    dst = jnp.pad(pi[0], (0, pad))
    src = jnp.pad(pi[1], (0, pad))
    src_shift = src[None, :] * NCH + jnp.arange(
        NCH, dtype=jnp.int32)[:, None]
    src_r = src_shift.reshape(NCH, NSUB, NB, 1, EB)
    dst_r = jnp.broadcast_to(dst.reshape(1, NSUB, NB, 1, EB),
                             (NCH, NSUB, NB, 1, EB))
    idx_hbm = jnp.concatenate([src_r, dst_r], axis=3)

    w = _mlp_call(rfv_p, d_p, W1, b1.reshape(1, -1), W2, b2.reshape(1, -1),
                  w3p, b3p.reshape(1, -1))

    x2 = X.reshape(N_ATOMS, 1152)
    u_perm, u_lin, xn = _prep_call(x2, L0, L1, L2)

    table = u_perm.reshape(NCH * N_ATOMS, 9, 16)
    msg_chunks = _sc_call(table, w, idx_hbm)
    msg_perm = msg_chunks.reshape(N_ATOMS, 1152)

    xout = _combine_call(msg_perm, u_lin, xn,
                         atomic_charges.reshape(N_ATOMS, 1), L3, L4, L5)
    return xout.reshape(N_ATOMS, N_FEAT, 3, 3)


# speedup vs baseline: 27.9951x; 1.1734x over previous
import functools

import jax
import jax.numpy as jnp
import numpy as np
from jax import lax
from jax.experimental import pallas as pl
from jax.experimental.pallas import tpu as pltpu
from jax.experimental.pallas import tpu_sc as plsc

N_ATOMS = 10000
N_FEAT = 128
N_RBF = 16
N_EDGES = 160000
CUTOFF = 5.0

NCORE = 2
NSUB = 16
NCH = 8
CPC = NCH // NCORE
EB = 64
NB = 158
EPT = NB * EB
E_PAD = EPT * NSUB
NPT = N_ATOMS // NSUB

_MLP_BLK = 2048
_NODE_BLK = 400

_PX = np.zeros((1152, 1152), np.float32)
for _f in range(128):
    for _k in range(9):
        _PX[_f * 9 + _k, _k * 128 + _f] = 1.0
_PT = np.zeros((1152, 1152), np.float32)
for _c in range(8):
    for _k in range(9):
        for _f in range(16):
            _PT[_k * 128 + _c * 16 + _f, _c * 144 + _k * 16 + _f] = 1.0
_PM = _PT.T.copy()
_PO = _PX.T.copy()


def _silu(x):
    return x * jax.nn.sigmoid(x)


def _mlp_body(rfv_ref, d_ref, w1_ref, b1_ref, w2_ref, b2_ref, w3_ref, b3_ref,
              out_ref):
    h = _silu(jnp.dot(rfv_ref[...], w1_ref[...],
                      preferred_element_type=jnp.float32) + b1_ref[...])
    h = _silu(jnp.dot(h, w2_ref[...],
                      preferred_element_type=jnp.float32) + b2_ref[...])
    h = _silu(jnp.dot(h, w3_ref[...],
                      preferred_element_type=jnp.float32) + b3_ref[...])
    d = d_ref[...]
    c = jnp.where(d < CUTOFF,
                  0.5 * (jnp.cos((jnp.pi / CUTOFF) * d) + 1.0), 0.0)
    out_ref[...] = h * c


def _mlp_call(rfv, d, w1, b1, w2, b2, w3p, b3p):
    grid = E_PAD // _MLP_BLK
    return pl.pallas_call(
        _mlp_body,
        grid=(grid,),
        in_specs=[
            pl.BlockSpec((_MLP_BLK, N_RBF), lambda i: (i, 0)),
            pl.BlockSpec((_MLP_BLK, 1), lambda i: (i, 0)),
            pl.BlockSpec((N_RBF, N_FEAT), lambda i: (0, 0)),
            pl.BlockSpec((1, N_FEAT), lambda i: (0, 0)),
            pl.BlockSpec((N_FEAT, 2 * N_FEAT), lambda i: (0, 0)),
            pl.BlockSpec((1, 2 * N_FEAT), lambda i: (0, 0)),
            pl.BlockSpec((2 * N_FEAT, 3 * N_FEAT), lambda i: (0, 0)),
            pl.BlockSpec((1, 3 * N_FEAT), lambda i: (0, 0)),
        ],
        out_specs=pl.BlockSpec((_MLP_BLK, 3 * N_FEAT), lambda i: (i, 0)),
        out_shape=jax.ShapeDtypeStruct((E_PAD, 3 * N_FEAT), jnp.float32),
        compiler_params=pltpu.CompilerParams(
            dimension_semantics=("parallel",)),
    )(rfv, d, w1, b1, w2, b2, w3p, b3p)


def _prep_body(x_ref, px_ref, pt_ref, l0_ref, l1_ref, l2_ref,
               uperm_ref, u_ref, xn_ref):
    xc = jnp.dot(x_ref[...], px_ref[...], preferred_element_type=jnp.float32)
    xs = [xc[:, k * N_FEAT:(k + 1) * N_FEAT] for k in range(9)]
    nrm = xs[0] * xs[0]
    for k in range(1, 9):
        nrm = nrm + xs[k] * xs[k]
    inv = 1.0 / (nrm + 1.0)
    xn = [v * inv for v in xs]
    for k in range(9):
        xn_ref[k] = xn[k]
    dg = (xn[0] + xn[4] + xn[8]) * (1.0 / 3.0)
    up = [dg,
          0.5 * (xn[1] - xn[3]), 0.5 * (xn[2] - xn[6]), 0.5 * (xn[5] - xn[7]),
          0.5 * (xn[1] + xn[3]), 0.5 * (xn[2] + xn[6]), 0.5 * (xn[5] + xn[7]),
          xn[0] - dg, xn[4] - dg]
    u = [jnp.dot(up[0], l0_ref[...], preferred_element_type=jnp.float32)]
    for k in range(1, 4):
        u.append(jnp.dot(up[k], l1_ref[...],
                         preferred_element_type=jnp.float32))
    for k in range(4, 9):
        u.append(jnp.dot(up[k], l2_ref[...],
                         preferred_element_type=jnp.float32))
    for k in range(9):
        u_ref[k] = u[k]
    ucat = jnp.concatenate(u, axis=1)
    uperm_ref[...] = jnp.dot(ucat, pt_ref[...],
                             preferred_element_type=jnp.float32)


def _prep_call(x2, l0, l1, l2):
    grid = N_ATOMS // _NODE_BLK
    lmat = lambda: pl.BlockSpec((N_FEAT, N_FEAT), lambda i: (0, 0))
    pmat = lambda: pl.BlockSpec((1152, 1152), lambda i: (0, 0))
    wide = lambda: pl.BlockSpec((_NODE_BLK, 1152), lambda i: (i, 0))
    tens = lambda: pl.BlockSpec((9, _NODE_BLK, N_FEAT), lambda i: (0, i, 0))
    return pl.pallas_call(
        _prep_body,
        grid=(grid,),
        in_specs=[wide(), pmat(), pmat(), lmat(), lmat(), lmat()],
        out_specs=[wide(), tens(), tens()],
        out_shape=[jax.ShapeDtypeStruct((N_ATOMS, 1152), jnp.float32),
                   jax.ShapeDtypeStruct((9, N_ATOMS, N_FEAT), jnp.float32),
                   jax.ShapeDtypeStruct((9, N_ATOMS, N_FEAT), jnp.float32)],
        compiler_params=pltpu.CompilerParams(
            dimension_semantics=("parallel",)),
    )(x2, jnp.asarray(_PX), jnp.asarray(_PT), l0, l1, l2)


def _sc_body(table_ref, w_ref, idx_ref, out_ref,
             ibuf0, ibuf1, gbuf0, gbuf1, wbuf0, wbuf1, zbuf, acc,
             isem0, isem1, gsem0, gsem1, wsem0, wsem1):
    c = lax.axis_index("c")
    s = lax.axis_index("s")
    e0 = s * EPT
    base = s * NPT

    def zrow(i, carry):
        for k in range(9):
            zbuf[i, pl.ds(k * 16, 16)] = jnp.zeros((16,), jnp.float32)
        return carry
    lax.fori_loop(0, 25, zrow, 0)

    for cl in range(CPC):
        chunk = c * CPC + cl
        for z in range(25):
            pltpu.sync_copy(zbuf, acc.at[pl.ds(base + z * 25, 25)])
        plsc.subcore_barrier()

        def i_idx(b, ib, sem):
            pltpu.async_copy(idx_ref.at[chunk, s, b], ib, sem)

        def w_idx(b, ib, sem):
            pltpu.make_async_copy(idx_ref.at[chunk, s, b], ib, sem).wait()

        def i_gw(b, ib, gb, gsem, wb, wsem):
            pltpu.async_copy(table_ref.at[ib.at[0]], gb, gsem)
            pltpu.async_copy(
                w_ref.at[pl.ds(e0 + b * EB, EB), pl.ds(chunk * 48, 48)],
                wb, wsem)

        def w_gw(b, ib, gb, gsem, wb, wsem):
            pltpu.make_async_copy(table_ref.at[ib.at[0]], gb, gsem).wait()
            pltpu.make_async_copy(
                w_ref.at[pl.ds(e0 + b * EB, EB), pl.ds(chunk * 48, 48)],
                wb, wsem).wait()

        def proc(gb, wb, ib):
            def edge(e8, cc):
                for u in range(8):
                    e = e8 * 8 + u
                    w0 = wb[e, pl.ds(0, 16)]
                    w1 = wb[e, pl.ds(16, 16)]
                    w2 = wb[e, pl.ds(32, 16)]
                    gb[e, pl.ds(0, 16)] = gb[e, pl.ds(0, 16)] * w0
                    for k in (1, 2, 3):
                        gb[e, pl.ds(k * 16, 16)] = (
                            gb[e, pl.ds(k * 16, 16)] * w1)
                    for k in (4, 5, 6, 7, 8):
                        gb[e, pl.ds(k * 16, 16)] = (
                            gb[e, pl.ds(k * 16, 16)] * w2)
                return cc
            lax.fori_loop(0, EB // 8, edge, 0)
            pltpu.sync_copy(gb, acc.at[ib.at[1]], add=True)

        i_idx(0, ibuf0, isem0)
        w_idx(0, ibuf0, isem0)
        i_gw(0, ibuf0, gbuf0, gsem0, wbuf0, wsem0)
        i_idx(1, ibuf1, isem1)

        def pair(t, carry):
            b0 = 2 * t
            w_idx(b0 + 1, ibuf1, isem1)
            i_gw(b0 + 1, ibuf1, gbuf1, gsem1, wbuf1, wsem1)
            w_gw(b0, ibuf0, gbuf0, gsem0, wbuf0, wsem0)
            proc(gbuf0, wbuf0, ibuf0)
            i_idx(b0 + 2, ibuf0, isem0)
            w_idx(b0 + 2, ibuf0, isem0)
            i_gw(b0 + 2, ibuf0, gbuf0, gsem0, wbuf0, wsem0)
            w_gw(b0 + 1, ibuf1, gbuf1, gsem1, wbuf1, wsem1)
            proc(gbuf1, wbuf1, ibuf1)
            i_idx(b0 + 3, ibuf1, isem1)
            return carry
        lax.fori_loop(0, NB // 2 - 1, pair, 0)

        w_idx(NB - 1, ibuf1, isem1)
        i_gw(NB - 1, ibuf1, gbuf1, gsem1, wbuf1, wsem1)
        w_gw(NB - 2, ibuf0, gbuf0, gsem0, wbuf0, wsem0)
        proc(gbuf0, wbuf0, ibuf0)
        w_gw(NB - 1, ibuf1, gbuf1, gsem1, wbuf1, wsem1)
        proc(gbuf1, wbuf1, ibuf1)

        plsc.subcore_barrier()
        pltpu.sync_copy(acc.at[pl.ds(base, NPT)],
                        out_ref.at[pl.ds(base, NPT), pl.ds(chunk * 144, 144)])
        plsc.subcore_barrier()


def _sc_call(table, w, idx_hbm):
    mesh = plsc.VectorSubcoreMesh(core_axis_name="c", subcore_axis_name="s")
    fn = pl.kernel(
        _sc_body,
        out_type=jax.ShapeDtypeStruct((N_ATOMS, 1152), jnp.float32),
        mesh=mesh,
        scratch_types=[
            pltpu.VMEM((2, EB), jnp.int32),
            pltpu.VMEM((2, EB), jnp.int32),
            pltpu.VMEM((EB, 144), jnp.float32),
            pltpu.VMEM((EB, 144), jnp.float32),
            pltpu.VMEM((EB, 48), jnp.float32),
            pltpu.VMEM((EB, 48), jnp.float32),
            pltpu.VMEM((25, 144), jnp.float32),
            pltpu.VMEM_SHARED((N_ATOMS, 144), jnp.float32),
            pltpu.SemaphoreType.DMA,
            pltpu.SemaphoreType.DMA,
            pltpu.SemaphoreType.DMA,
            pltpu.SemaphoreType.DMA,
            pltpu.SemaphoreType.DMA,
            pltpu.SemaphoreType.DMA,
        ],
        compiler_params=pltpu.CompilerParams(use_tc_tiling_on_sc=False),
    )
    return fn(table, w, idx_hbm)


def _decompress(u):
    return [u[0] + u[7], u[1] + u[4], u[2] + u[5],
            u[4] - u[1], u[0] + u[8], u[3] + u[6],
            u[5] - u[2], u[6] - u[3], u[0] - u[7] - u[8]]


def _combine_body(msg_ref, u_ref, xn_ref, q_ref, pm_ref, l3_ref, l4_ref,
                  l5_ref, po_ref, out_ref):
    mc = jnp.dot(msg_ref[...], pm_ref[...],
                 preferred_element_type=jnp.float32)
    m = _decompress([mc[:, k * N_FEAT:(k + 1) * N_FEAT] for k in range(9)])
    y = _decompress([u_ref[k] for k in range(9)])
    t = [None] * 9
    for i in range(3):
        for k in range(3):
            acc = None
            for j in range(3):
                term = m[3 * i + j] * y[3 * j + k] + y[3 * i + j] * m[3 * j + k]
                acc = term if acc is None else acc + term
            t[3 * i + k] = acc
    nrm = t[0] * t[0]
    for k in range(1, 9):
        nrm = nrm + t[k] * t[k]
    inv = 1.0 / (nrm + 1.0)
    dg = (t[0] + t[4] + t[8]) * (1.0 / 3.0)
    v = [dg,
         0.5 * (t[1] - t[3]), 0.5 * (t[2] - t[6]), 0.5 * (t[5] - t[7]),
         0.5 * (t[1] + t[3]), 0.5 * (t[2] + t[6]), 0.5 * (t[5] + t[7]),
         t[0] - dg, t[4] - dg]
    v = [vi * inv for vi in v]
    w = [jnp.dot(v[0], l3_ref[...], preferred_element_type=jnp.float32)]
    for k in range(1, 4):
        w.append(jnp.dot(v[k], l4_ref[...],
                         preferred_element_type=jnp.float32))
    for k in range(4, 9):
        w.append(jnp.dot(v[k], l5_ref[...],
                         preferred_element_type=jnp.float32))
    dx = _decompress(w)
    f = 1.0 + 0.1 * q_ref[...]
    outc = [None] * 9
    for i in range(3):
        for k in range(3):
            acc = None
            for j in range(3):
                term = dx[3 * i + j] * dx[3 * j + k]
                acc = term if acc is None else acc + term
            outc[3 * i + k] = (xn_ref[3 * i + k] + dx[3 * i + k] + f * acc)
    ocat = jnp.concatenate(outc, axis=1)
    out_ref[...] = jnp.dot(ocat, po_ref[...],
                           preferred_element_type=jnp.float32)


def _combine_call(msg_perm, u_lin, xn, q, l3, l4, l5):
    grid = N_ATOMS // _NODE_BLK
    lmat = lambda: pl.BlockSpec((N_FEAT, N_FEAT), lambda i: (0, 0))
    pmat = lambda: pl.BlockSpec((1152, 1152), lambda i: (0, 0))
    wide = lambda: pl.BlockSpec((_NODE_BLK, 1152), lambda i: (i, 0))
    tens = lambda: pl.BlockSpec((9, _NODE_BLK, N_FEAT), lambda i: (0, i, 0))
    return pl.pallas_call(
        _combine_body,
        grid=(grid,),
        in_specs=[wide(), tens(), tens(),
                  pl.BlockSpec((_NODE_BLK, 1), lambda i: (i, 0)),
                  pmat(), lmat(), lmat(), lmat(), pmat()],
        out_specs=wide(),
        out_shape=jax.ShapeDtypeStruct((N_ATOMS, 1152), jnp.float32),
        compiler_params=pltpu.CompilerParams(
            dimension_semantics=("parallel",)),
    )(msg_perm, u_lin, xn, q, jnp.asarray(_PM), l3, l4, l5,
      jnp.asarray(_PO))


def kernel(X, pair_indices, d_ij, radial_feature_vector, atomic_charges,
           W1, b1, W2, b2, W3, b3, L0, L1, L2, L3, L4, L5):
    cols = jnp.arange(3 * N_FEAT)
    cchunk = cols // 48
    rem = cols % 48
    chan = rem // 16
    feat = rem % 16
    src_col = (cchunk * 16 + feat) * 3 + chan
    w3p = W3[:, src_col]
    b3p = b3[src_col]

    pad = E_PAD - N_EDGES
    rfv_p = jnp.pad(radial_feature_vector, ((0, pad), (0, 0)))
    d_p = jnp.pad(d_ij, ((0, pad), (0, 0)), constant_values=CUTOFF)
    pi = pair_indices.astype(jnp.int32)
    dst = jnp.pad(pi[0], (0, pad))
    src = jnp.pad(pi[1], (0, pad))
    src_shift = src[None, :] * NCH + jnp.arange(
        NCH, dtype=jnp.int32)[:, None]
    src_r = src_shift.reshape(NCH, NSUB, NB, 1, EB)
    dst_r = jnp.broadcast_to(dst.reshape(1, NSUB, NB, 1, EB),
                             (NCH, NSUB, NB, 1, EB))
    idx_hbm = jnp.concatenate([src_r, dst_r], axis=3)

    w = _mlp_call(rfv_p, d_p, W1, b1.reshape(1, -1), W2, b2.reshape(1, -1),
                  w3p, b3p.reshape(1, -1))

    x2 = X.reshape(N_ATOMS, 1152)
    u_perm, u_lin, xn = _prep_call(x2, L0, L1, L2)

    table = u_perm.reshape(NCH * N_ATOMS, 144)
    msg_perm = _sc_call(table, w, idx_hbm)

    xout = _combine_call(msg_perm, u_lin, xn,
                         atomic_charges.reshape(N_ATOMS, 1), L3, L4, L5)
    return xout.reshape(N_ATOMS, N_FEAT, 3, 3)
